# Initial kernel scaffold; baseline (speedup 1.0000x reference)
import jax
import jax.numpy as jnp
from jax.experimental import pallas as pl


def kernel(x, rot2, centroids):
    raise NotImplementedError("write your pallas kernel here")


# speedup vs baseline: 13.4749x; 13.4749x over previous
import functools
import jax
import jax.numpy as jnp
from jax.experimental import pallas as pl
from jax.experimental.pallas import tpu as pltpu

_D = 256
_N_LEVELS = 16


def _tc_body(scal_ref, x_ref, ab_ref, o_ref):
    x = x_ref[...]
    n2 = jnp.sum(x * x, axis=1, keepdims=True)
    norm = jnp.maximum(jnp.sqrt(n2), 1e-8)
    inv = 1.0 / norm

    lane = jax.lax.broadcasted_iota(jnp.int32, (1, _D), 1)
    even = (lane % 2) == 0

    a = ab_ref[0:1, :]
    b = ab_ref[1:2, :]

    xs = jnp.where(even, jnp.roll(x, -1, axis=1), jnp.roll(x, 1, axis=1))
    f = (a * x + b * xs) * inv

    q = jnp.full(f.shape, scal_ref[0], dtype=jnp.float32)
    for k in range(_N_LEVELS - 1):
        q = q + jnp.where(f > scal_ref[1 + k], scal_ref[16 + k], 0.0)

    qs = jnp.where(even, jnp.roll(q, -1, axis=1), jnp.roll(q, 1, axis=1))
    o_ref[...] = (a * q + b * qs) * norm


def _tc_quant(x, ab, scal, bm, interpret=False):
    B = x.shape[0]
    grid = (B // bm,)
    return pl.pallas_call(
        _tc_body,
        grid=grid,
        in_specs=[
            pl.BlockSpec(memory_space=pltpu.SMEM),
            pl.BlockSpec((bm, _D), lambda i: (i, 0)),
            pl.BlockSpec((2, _D), lambda i: (0, 0)),
        ],
        out_specs=pl.BlockSpec((bm, _D), lambda i: (i, 0)),
        out_shape=jax.ShapeDtypeStruct((B, _D), jnp.float32),
        interpret=interpret,
    )(scal, x, ab)


@functools.partial(jax.jit, static_argnames=("interpret",))
def kernel(x, rot2, centroids, interpret=False):
    c = rot2[:, 0]
    s = rot2[:, 1]
    a = jnp.repeat(c, 2)
    b = jnp.stack([-s, s], axis=-1).reshape(-1)
    ab = jnp.stack([a, b], axis=0)
    mids = 0.5 * (centroids[1:] + centroids[:-1])
    dlt = centroids[1:] - centroids[:-1]
    scal = jnp.concatenate(
        [centroids[0:1], mids, dlt, jnp.zeros((1,), jnp.float32)]
    )
    return _tc_quant(x, ab, scal, bm=512, interpret=interpret)
